# baseline (speedup 1.0000x reference)
import jax
import jax.numpy as jnp
from jax.experimental import pallas as pl
from jax.experimental.pallas import tpu as pltpu

_CHUNK = 40
_NBUF = 10


def _gcn_kernel(x_ref, adj_ref, w_ref, b_ref, out_ref, bufs, support, sems):
    n = x_ref.shape[0]
    nchunk = n // _CHUNK

    def chunk_copy(i, slot):
        return pltpu.make_async_copy(
            adj_ref.at[pl.ds(i * _CHUNK, _CHUNK), :],
            bufs.at[slot],
            sems.at[slot],
        )

    for j in range(_NBUF - 1):
        chunk_copy(j, j).start()

    support[...] = jnp.dot(
        x_ref[...], w_ref[...], preferred_element_type=jnp.float32
    )

    def body(i, carry):
        nxt = i + _NBUF - 1

        @pl.when(nxt < nchunk)
        def _():
            chunk_copy(nxt, jax.lax.rem(nxt, _NBUF)).start()

        slot = jax.lax.rem(i, _NBUF)
        chunk_copy(i, slot).wait()

        out_ref[pl.ds(i * _CHUNK, _CHUNK), :] = (
            jnp.dot(bufs[slot], support[...], preferred_element_type=jnp.float32)
            + b_ref[...]
        )
        return carry

    jax.lax.fori_loop(0, nchunk, body, 0)


def kernel(input, adj, W, b):
    n, d_in = input.shape
    d_out = W.shape[1]
    b2 = b.reshape(1, d_out)
    return pl.pallas_call(
        _gcn_kernel,
        in_specs=[
            pl.BlockSpec(memory_space=pltpu.MemorySpace.VMEM),
            pl.BlockSpec(memory_space=pltpu.MemorySpace.HBM),
            pl.BlockSpec(memory_space=pltpu.MemorySpace.VMEM),
            pl.BlockSpec(memory_space=pltpu.MemorySpace.VMEM),
        ],
        out_specs=pl.BlockSpec(memory_space=pltpu.MemorySpace.VMEM),
        out_shape=jax.ShapeDtypeStruct((n, d_out), jnp.float32),
        scratch_shapes=[
            pltpu.VMEM((_NBUF, _CHUNK, n), jnp.float32),
            pltpu.VMEM((n, d_out), jnp.float32),
            pltpu.SemaphoreType.DMA((_NBUF,)),
        ],
    )(input, adj, W, b2)


# speedup vs baseline: 1.1743x; 1.1743x over previous
import jax
import jax.numpy as jnp
from jax.experimental import pallas as pl
from jax.experimental.pallas import tpu as pltpu

_GROUP = 400
_SUB = 10
_CHUNK = _GROUP // _SUB


def _gcn_kernel(x_ref, adj_ref, w_ref, b_ref, out_ref, bufs, support, sems):
    n = x_ref.shape[0]
    ngroup = n // _GROUP

    def sub_copy(g, s, slot):
        return pltpu.make_async_copy(
            adj_ref.at[pl.ds(g * _GROUP + s * _CHUNK, _CHUNK), :],
            bufs.at[slot, pl.ds(s * _CHUNK, _CHUNK), :],
            sems.at[slot],
        )

    for s in range(_SUB):
        sub_copy(0, s, 0).start()

    support[...] = jnp.dot(
        x_ref[...], w_ref[...], preferred_element_type=jnp.float32
    )

    def body(g, carry):
        slot = jax.lax.rem(g, 2)

        @pl.when(g + 1 < ngroup)
        def _():
            nslot = jax.lax.rem(g + 1, 2)
            for s in range(_SUB):
                sub_copy(g + 1, s, nslot).start()

        for s in range(_SUB):
            sub_copy(g, s, slot).wait()

        out_ref[pl.ds(g * _GROUP, _GROUP), :] = (
            jnp.dot(bufs[slot], support[...], preferred_element_type=jnp.float32)
            + b_ref[...]
        )
        return carry

    jax.lax.fori_loop(0, ngroup, body, 0)


def kernel(input, adj, W, b):
    n, d_in = input.shape
    d_out = W.shape[1]
    b2 = b.reshape(1, d_out)
    return pl.pallas_call(
        _gcn_kernel,
        in_specs=[
            pl.BlockSpec(memory_space=pltpu.MemorySpace.VMEM),
            pl.BlockSpec(memory_space=pltpu.MemorySpace.HBM),
            pl.BlockSpec(memory_space=pltpu.MemorySpace.VMEM),
            pl.BlockSpec(memory_space=pltpu.MemorySpace.VMEM),
        ],
        out_specs=pl.BlockSpec(memory_space=pltpu.MemorySpace.VMEM),
        out_shape=jax.ShapeDtypeStruct((n, d_out), jnp.float32),
        scratch_shapes=[
            pltpu.VMEM((2, _GROUP, n), jnp.float32),
            pltpu.VMEM((n, d_out), jnp.float32),
            pltpu.SemaphoreType.DMA((2,)),
        ],
    )(input, adj, W, b2)
